# baseline (speedup 1.0000x reference)
import functools

import jax
import jax.numpy as jnp
from jax import lax
from jax.experimental import pallas as pl
from jax.experimental.pallas import tpu as pltpu
from jax.experimental.pallas import tpu_sc as plsc

BATCH = 16384
LANES = 128

_info = plsc.get_sparse_core_info()
NC, NS = _info.num_cores, _info.num_subcores
NW = NC * NS
BPW = BATCH // NW
CHUNK = 128
NCHUNK = BPW // CHUNK
PCHUNK = CHUNK // 2

_sc_mesh = plsc.VectorSubcoreMesh(core_axis_name="c", subcore_axis_name="s")


@functools.partial(
    pl.kernel,
    out_type=jax.ShapeDtypeStruct((BATCH // 2, LANES), jnp.float32),
    mesh=_sc_mesh,
    scratch_types=[
        pltpu.VMEM((BPW,), jnp.int32),
        pltpu.VMEM((CHUNK, LANES), jnp.float32),
        pltpu.VMEM((CHUNK, LANES), jnp.float32),
        pltpu.VMEM((PCHUNK, LANES), jnp.float32),
        pltpu.VMEM((PCHUNK, LANES), jnp.float32),
        pltpu.SemaphoreType.DMA,
        pltpu.SemaphoreType.DMA,
    ],
)
def _sc_gather(sub_idx_hbm, sub_tab_hbm, x_out,
               sub_idx_v, sub_rows0, sub_rows1, pack0, pack1,
               sem_g, sem_w):
    wid = lax.axis_index("s") * NC + lax.axis_index("c")
    base = wid * BPW
    pbase = wid * (BPW // 2)

    pltpu.sync_copy(sub_idx_hbm.at[pl.ds(base, BPW)], sub_idx_v)

    sub_bufs = [sub_rows0, sub_rows1]
    packs = [pack0, pack1]
    gathers = [None] * NCHUNK
    writes = [None] * NCHUNK

    def fire(j):
        isl = pl.ds(j * CHUNK, CHUNK)
        gathers[j] = pltpu.async_copy(
            sub_tab_hbm.at[sub_idx_v.at[isl]], sub_bufs[j % 2], sem_g)

    fire(0)
    for j in range(NCHUNK):
        if j + 1 < NCHUNK:
            if j - 1 >= 0:
                writes[j - 1].wait()
            fire(j + 1)
        gathers[j].wait()
        b = j % 2
        sb, pb = sub_bufs[b], packs[b]

        def pack_row(r, _):
            r0 = 2 * r
            r1 = 2 * r + 1
            for k in (0, 1):
                sl = pl.ds(k * 16, 16)
                pb.at[r][sl] = sb.at[r0][sl]
                pb.at[r][pl.ds(64 + k * 16, 16)] = sb.at[r1][sl]
            return 0

        lax.fori_loop(0, PCHUNK, pack_row, 0)
        writes[j] = pltpu.async_copy(
            pb, x_out.at[pl.ds(pbase + j * PCHUNK, PCHUNK)], sem_w)
    writes[NCHUNK - 2].wait()
    writes[NCHUNK - 1].wait()


def _mlp_body(x_ref, tid_ref, cid_ref, tt_ref, ct_ref,
              w1_ref, b1_ref, w2_ref, b2_ref, w3_ref, b3_ref, out_ref):
    blk = x_ref[...]
    xc = jnp.concatenate([blk[:, :64], blk[:, 64:]], axis=0)
    x = xc[:, :32].astype(jnp.bfloat16)
    w1 = w1_ref[...].astype(jnp.bfloat16)
    tt = tt_ref[...].astype(jnp.bfloat16)
    ct = ct_ref[...].astype(jnp.bfloat16)
    w1t = jnp.dot(tt, w1[32:48], preferred_element_type=jnp.float32)
    w1c = jnp.dot(ct, w1[48:56], preferred_element_type=jnp.float32)
    iota = lax.broadcasted_iota(jnp.int32, (1, 16), 1)
    oh_t = (tid_ref[...] == iota).astype(jnp.bfloat16)
    oh_c = (cid_ref[...] == iota).astype(jnp.bfloat16)
    h = (jnp.dot(x, w1[:32], preferred_element_type=jnp.float32)
         + jnp.dot(oh_t, w1t.astype(jnp.bfloat16),
                   preferred_element_type=jnp.float32)
         + jnp.dot(oh_c, w1c.astype(jnp.bfloat16),
                   preferred_element_type=jnp.float32)
         + b1_ref[...])
    h = jnp.maximum(h, 0.0).astype(jnp.bfloat16)
    w2 = w2_ref[...].astype(jnp.bfloat16)
    w3 = w3_ref[...].astype(jnp.bfloat16)
    h = jnp.dot(h, w2, preferred_element_type=jnp.float32) + b2_ref[...]
    h = jnp.maximum(h, 0.0).astype(jnp.bfloat16)
    y = jnp.dot(h, w3, preferred_element_type=jnp.float32) + b3_ref[...]
    n = y.shape[0] // 2
    out_ref[...] = jnp.stack([y[:n], y[n:]], axis=1).reshape(2 * n, 64)


P_BLK = 2048


def _mlp(x, tid, cid, tt, ct, w1, b1, w2, b2, w3, b3):
    full = lambda shape: pl.BlockSpec(shape, lambda i: tuple(0 for _ in shape))
    return pl.pallas_call(
        _mlp_body,
        grid=(BATCH // 2 // P_BLK,),
        in_specs=[
            pl.BlockSpec((P_BLK, LANES), lambda i: (i, 0)),
            pl.BlockSpec((2 * P_BLK, 1), lambda i: (i, 0)),
            pl.BlockSpec((2 * P_BLK, 1), lambda i: (i, 0)),
            full((16, 16)),
            full((16, 8)),
            full((56, 512)),
            full((512,)),
            full((512, 128)),
            full((128,)),
            full((128, 64)),
            full((64,)),
        ],
        out_specs=pl.BlockSpec((2 * P_BLK, 64), lambda i: (i, 0)),
        out_shape=jax.ShapeDtypeStruct((BATCH, 64), jnp.float32),
    )(x, tid, cid, tt, ct, w1, b1, w2, b2, w3, b3)


def kernel(subcontractor_id, primary_trade_id, certification_id,
           sub_table, trade_table, cert_table,
           W1, b1, W2, b2, W3, b3):
    sub_idx = subcontractor_id.astype(jnp.int32)
    sub_tab_p = jnp.pad(sub_table, ((0, 0), (0, LANES - 32)))

    def perm(idx):
        a = idx.astype(jnp.int32).reshape(-1, P_BLK, 2)
        return a.transpose(0, 2, 1).reshape(BATCH, 1)

    tid = perm(primary_trade_id)
    cid = perm(certification_id)
    tt = jnp.pad(trade_table, ((0, 5), (0, 0)))
    ct = jnp.pad(cert_table, ((0, 7), (0, 0)))

    x = _sc_gather(sub_idx, sub_tab_p)
    return _mlp(x, tid, cid, tt, ct, W1, b1, W2, b2, W3, b3)


# speedup vs baseline: 1.2821x; 1.2821x over previous
import functools

import jax
import jax.numpy as jnp
from jax import lax
from jax.experimental import pallas as pl
from jax.experimental.pallas import tpu as pltpu
from jax.experimental.pallas import tpu_sc as plsc

BATCH = 16384
LANES = 128

_info = plsc.get_sparse_core_info()
NC, NS = _info.num_cores, _info.num_subcores
NW = NC * NS
BPW = BATCH // NW
CHUNK = 128
NCHUNK = BPW // CHUNK
PCHUNK = CHUNK // 2

_sc_mesh = plsc.VectorSubcoreMesh(core_axis_name="c", subcore_axis_name="s")


@functools.partial(
    pl.kernel,
    out_type=jax.ShapeDtypeStruct((BATCH // 2, LANES), jnp.float32),
    mesh=_sc_mesh,
    scratch_types=[
        pltpu.VMEM((BPW,), jnp.int32),
        pltpu.VMEM((CHUNK, LANES), jnp.float32),
        pltpu.VMEM((CHUNK, LANES), jnp.float32),
        pltpu.VMEM((PCHUNK, LANES), jnp.float32),
        pltpu.VMEM((PCHUNK, LANES), jnp.float32),
        pltpu.SemaphoreType.DMA,
        pltpu.SemaphoreType.DMA,
    ],
)
def _sc_gather(sub_idx_hbm, sub_tab_hbm, x_out,
               sub_idx_v, sub_rows0, sub_rows1, pack0, pack1,
               sem_g, sem_w):
    wid = lax.axis_index("s") * NC + lax.axis_index("c")
    base = wid * BPW
    pbase = wid * (BPW // 2)

    pltpu.sync_copy(sub_idx_hbm.at[pl.ds(base, BPW)], sub_idx_v)

    sub_bufs = [sub_rows0, sub_rows1]
    packs = [pack0, pack1]
    gathers = [None] * NCHUNK
    writes = [None] * NCHUNK

    def fire(j):
        isl = pl.ds(j * CHUNK, CHUNK)
        gathers[j] = pltpu.async_copy(
            sub_tab_hbm.at[sub_idx_v.at[isl]], sub_bufs[j % 2], sem_g)

    fire(0)
    for j in range(NCHUNK):
        if j + 1 < NCHUNK:
            if j - 1 >= 0:
                writes[j - 1].wait()
            fire(j + 1)
        gathers[j].wait()
        b = j % 2
        sb, pb = sub_bufs[b], packs[b]

        def pack_row(r, _):
            r0 = 2 * r
            r1 = 2 * r + 1
            for k in (0, 1):
                sl = pl.ds(k * 16, 16)
                pb.at[r][sl] = sb.at[r0][sl]
                pb.at[r][pl.ds(64 + k * 16, 16)] = sb.at[r1][sl]
            return 0

        lax.fori_loop(0, PCHUNK, pack_row, 0)
        writes[j] = pltpu.async_copy(
            pb, x_out.at[pl.ds(pbase + j * PCHUNK, PCHUNK)], sem_w)
    writes[NCHUNK - 2].wait()
    writes[NCHUNK - 1].wait()


def _mlp_body(x_ref, oh_ref, tt_ref, ct_ref,
              w1_ref, b1_ref, w2_ref, b2_ref, w3_ref, b3_ref, out_ref):
    blk = x_ref[...]
    xc = jnp.concatenate([blk[:, :64], blk[:, 64:]], axis=0)
    x = xc[:, :32].astype(jnp.bfloat16)
    w1 = w1_ref[...].astype(jnp.bfloat16)
    tt = tt_ref[...].astype(jnp.bfloat16)
    ct = ct_ref[...].astype(jnp.bfloat16)
    w1t = jnp.dot(tt, w1[32:48], preferred_element_type=jnp.float32)
    w1c = jnp.dot(ct, w1[48:56], preferred_element_type=jnp.float32)
    ohb = oh_ref[...]
    ohc = jnp.concatenate([ohb[:, :64], ohb[:, 64:]], axis=0)
    oh_t = ohc[:, :16]
    oh_c = ohc[:, 16:32]
    h = (jnp.dot(x, w1[:32], preferred_element_type=jnp.float32)
         + jnp.dot(oh_t, w1t.astype(jnp.bfloat16),
                   preferred_element_type=jnp.float32)
         + jnp.dot(oh_c, w1c.astype(jnp.bfloat16),
                   preferred_element_type=jnp.float32)
         + b1_ref[...])
    h = jnp.maximum(h, 0.0).astype(jnp.bfloat16)
    w2 = w2_ref[...].astype(jnp.bfloat16)
    w3 = w3_ref[...].astype(jnp.bfloat16)
    h = jnp.dot(h, w2, preferred_element_type=jnp.float32) + b2_ref[...]
    h = jnp.maximum(h, 0.0).astype(jnp.bfloat16)
    y = jnp.dot(h, w3, preferred_element_type=jnp.float32) + b3_ref[...]
    n = y.shape[0] // 2
    out_ref[...] = jnp.stack([y[:n], y[n:]], axis=1).reshape(2 * n, 64)


P_BLK = 2048


def _mlp(x, oh, tt, ct, w1, b1, w2, b2, w3, b3):
    full = lambda shape: pl.BlockSpec(shape, lambda i: tuple(0 for _ in shape))
    return pl.pallas_call(
        _mlp_body,
        grid=(BATCH // 2 // P_BLK,),
        in_specs=[
            pl.BlockSpec((P_BLK, LANES), lambda i: (i, 0)),
            pl.BlockSpec((P_BLK, LANES), lambda i: (i, 0)),
            full((16, 16)),
            full((16, 8)),
            full((56, 512)),
            full((512,)),
            full((512, 128)),
            full((128,)),
            full((128, 64)),
            full((64,)),
        ],
        out_specs=pl.BlockSpec((2 * P_BLK, 64), lambda i: (i, 0)),
        out_shape=jax.ShapeDtypeStruct((BATCH, 64), jnp.float32),
    )(x, oh, tt, ct, w1, b1, w2, b2, w3, b3)


def kernel(subcontractor_id, primary_trade_id, certification_id,
           sub_table, trade_table, cert_table,
           W1, b1, W2, b2, W3, b3):
    sub_idx = subcontractor_id.astype(jnp.int32)
    sub_tab_p = jnp.pad(sub_table, ((0, 0), (0, LANES - 32)))

    iota16 = jnp.arange(16)
    oh_row = jnp.concatenate([
        (primary_trade_id[:, None] == iota16).astype(jnp.bfloat16),
        (certification_id[:, None] == iota16).astype(jnp.bfloat16),
        jnp.zeros((BATCH, 32), jnp.bfloat16),
    ], axis=1)
    oh = oh_row.reshape(BATCH // 2, LANES)

    tt = jnp.pad(trade_table, ((0, 5), (0, 0)))
    ct = jnp.pad(cert_table, ((0, 7), (0, 0)))

    x = _sc_gather(sub_idx, sub_tab_p)
    return _mlp(x, oh, tt, ct, W1, b1, W2, b2, W3, b3)
